# speedup vs baseline: 31.9677x; 31.9677x over previous
import jax
import jax.numpy as jnp
from jax.experimental import pallas as pl
from jax.experimental.pallas import tpu as pltpu

_S0 = (0, 5, 6, 11, 12)
_S1 = (7, 8, 13, 14)
_S2 = (9, 10, 15, 16)
_SUBSETS = (_S0, _S1, _S2)
_CROSS_COUNT = (float(len(_S1)), float(len(_S0) + len(_S2)), float(len(_S1)))


def _body(kx_ref, ky_ref, s_ref, W0_ref, W1_ref, W2_ref, b0_ref, b1_ref,
          b2_ref, WeAB_ref, be_ref, Wp_ref, bp_ref, out_ref):
    kx = kx_ref[...]
    ky = ky_ref[...]
    s = s_ref[...]
    nb = kx.shape[0]

    xmin = jnp.min(kx, axis=1, keepdims=True)
    xmax = jnp.max(kx, axis=1, keepdims=True)
    ymin = jnp.min(ky, axis=1, keepdims=True)
    ymax = jnp.max(ky, axis=1, keepdims=True)
    xn = (kx - xmin) / (xmax - xmin + 1e-6)
    yn = (ky - ymin) / (ymax - ymin + 1e-6)

    w_refs = (W0_ref, W1_ref, W2_ref)
    b_refs = (b0_ref, b1_ref, b2_ref)
    be = be_ref[...]

    totals = []
    feats = []
    for li, S in enumerate(_SUBSETS):
        W = w_refs[li][...]
        b = b_refs[li][...]
        ms = []
        for u in S:
            xu = xn[:, u:u + 1]
            yu = yn[:, u:u + 1]
            su = s[:, u:u + 1]
            h = jnp.maximum(
                xu * W[0:1, :] + yu * W[1:2, :] + su * W[2:3, :] + b, 0.0)
            ms.append(h * su)
        total = ms[0]
        for m in ms[1:]:
            total = total + m
        totals.append(total)
        for idx, u in enumerate(S):
            feats.append((total - ms[idx]) * s[:, u:u + 1])

    fcat = jnp.concatenate(feats, axis=0)
    uv = jnp.dot(fcat, WeAB_ref[...],
                 preferred_element_type=jnp.float32)

    pooled_parts = []
    off = 0
    for li, S in enumerate(_SUBSETS):
        n = len(S)
        us = []
        vs = []
        for idx in range(n):
            blk = uv[(off + idx) * nb:(off + idx + 1) * nb, :]
            us.append(blk[:, :64])
            vs.append(blk[:, 64:])
        off += n
        maxall = vs[0]
        for v in vs[1:]:
            maxall = jnp.maximum(maxall, v)
        zsum = None
        for idx in range(n):
            om = None
            for j2 in range(n):
                if j2 == idx:
                    continue
                om = vs[j2] if om is None else jnp.maximum(om, vs[j2])
            z = jnp.maximum(us[idx] + be + jnp.maximum(om, 0.0), 0.0)
            zsum = z if zsum is None else zsum + z
        zc = jnp.maximum(be + maxall, 0.0)
        mean_z = (zsum + _CROSS_COUNT[li] * zc) * (1.0 / 17.0)
        mean_h = totals[li] * ((len(S) - 1) / 17.0)
        pooled_parts.append(mean_h)
        pooled_parts.append(mean_z)

    pooled = jnp.concatenate(pooled_parts, axis=1)
    out_ref[...] = (jnp.dot(pooled, Wp_ref[...],
                            preferred_element_type=jnp.float32)
                    + bp_ref[...])


def _run(kx, ky, scores, W0, W1, W2, b0r, b1r, b2r, WeAB, ber, Wp, bpr,
         block_n, interpret=False):
    n = kx.shape[0]
    grid = (n // block_n,)
    rep2 = lambda i: (0, 0)
    return pl.pallas_call(
        _body,
        grid=grid,
        in_specs=[
            pl.BlockSpec((block_n, 17), lambda i: (i, 0)),
            pl.BlockSpec((block_n, 17), lambda i: (i, 0)),
            pl.BlockSpec((block_n, 17), lambda i: (i, 0)),
            pl.BlockSpec((3, 64), rep2),
            pl.BlockSpec((3, 64), rep2),
            pl.BlockSpec((3, 64), rep2),
            pl.BlockSpec((1, 64), rep2),
            pl.BlockSpec((1, 64), rep2),
            pl.BlockSpec((1, 64), rep2),
            pl.BlockSpec((64, 128), rep2),
            pl.BlockSpec((1, 64), rep2),
            pl.BlockSpec((384, 128), rep2),
            pl.BlockSpec((1, 128), rep2),
        ],
        out_specs=pl.BlockSpec((block_n, 128), lambda i: (i, 0)),
        out_shape=jax.ShapeDtypeStruct((n, 128), jnp.float32),
        compiler_params=pltpu.CompilerParams(
            dimension_semantics=("parallel",)),
        interpret=interpret,
    )(kx, ky, scores, W0, W1, W2, b0r, b1r, b2r, WeAB, ber, Wp, bpr)


def kernel(keypoints, scores, W0, b0, W1, b1, W2, b2, We, be, Wp, bp,
           interpret=False):
    kx = keypoints[:, :, 0]
    ky = keypoints[:, :, 1]
    WeAB = jnp.concatenate([We[:64] - We[64:], We[64:]], axis=1)
    return _run(kx, ky, scores, W0, W1, W2,
                b0.reshape(1, 64), b1.reshape(1, 64), b2.reshape(1, 64),
                WeAB, be.reshape(1, 64), Wp, bp.reshape(1, 128),
                block_n=256, interpret=interpret)


# speedup vs baseline: 32.3088x; 1.0107x over previous
import jax
import jax.numpy as jnp
from jax.experimental import pallas as pl
from jax.experimental.pallas import tpu as pltpu

_S0 = (0, 5, 6, 11, 12)
_S1 = (7, 8, 13, 14)
_S2 = (9, 10, 15, 16)
_SUBSETS = (_S0, _S1, _S2)
_CROSS_COUNT = (float(len(_S1)), float(len(_S0) + len(_S2)), float(len(_S1)))


def _body(kx_ref, ky_ref, s_ref, W0_ref, W1_ref, W2_ref, b0_ref, b1_ref,
          b2_ref, WeAB_ref, be_ref, Wp_ref, bp_ref, out_ref):
    kx = kx_ref[...]
    ky = ky_ref[...]
    s = s_ref[...]
    nb = kx.shape[0]

    xmin = jnp.min(kx, axis=1, keepdims=True)
    xmax = jnp.max(kx, axis=1, keepdims=True)
    ymin = jnp.min(ky, axis=1, keepdims=True)
    ymax = jnp.max(ky, axis=1, keepdims=True)
    xn = (kx - xmin) / (xmax - xmin + 1e-6)
    yn = (ky - ymin) / (ymax - ymin + 1e-6)

    w_refs = (W0_ref, W1_ref, W2_ref)
    b_refs = (b0_ref, b1_ref, b2_ref)
    be = be_ref[...]

    totals = []
    feats = []
    for li, S in enumerate(_SUBSETS):
        W = w_refs[li][...]
        b = b_refs[li][...]
        ms = []
        for u in S:
            xu = xn[:, u:u + 1]
            yu = yn[:, u:u + 1]
            su = s[:, u:u + 1]
            h = jnp.maximum(
                xu * W[0:1, :] + yu * W[1:2, :] + su * W[2:3, :] + b, 0.0)
            ms.append(h * su)
        total = ms[0]
        for m in ms[1:]:
            total = total + m
        totals.append(total)
        for idx, u in enumerate(S):
            feats.append((total - ms[idx]) * s[:, u:u + 1])

    fcat = jnp.concatenate(feats, axis=0)
    uv = jnp.dot(fcat.astype(jnp.bfloat16), WeAB_ref[...],
                 preferred_element_type=jnp.float32)

    pooled_parts = []
    off = 0
    for li, S in enumerate(_SUBSETS):
        n = len(S)
        us = []
        vs = []
        for idx in range(n):
            blk = uv[(off + idx) * nb:(off + idx + 1) * nb, :]
            us.append(blk[:, :64])
            vs.append(blk[:, 64:])
        off += n
        maxall = vs[0]
        for v in vs[1:]:
            maxall = jnp.maximum(maxall, v)
        zsum = None
        for idx in range(n):
            om = None
            for j2 in range(n):
                if j2 == idx:
                    continue
                om = vs[j2] if om is None else jnp.maximum(om, vs[j2])
            z = jnp.maximum(us[idx] + be + jnp.maximum(om, 0.0), 0.0)
            zsum = z if zsum is None else zsum + z
        zc = jnp.maximum(be + maxall, 0.0)
        mean_z = (zsum + _CROSS_COUNT[li] * zc) * (1.0 / 17.0)
        mean_h = totals[li] * ((len(S) - 1) / 17.0)
        pooled_parts.append(mean_h)
        pooled_parts.append(mean_z)

    pooled = jnp.concatenate(pooled_parts, axis=1)
    out_ref[...] = (jnp.dot(pooled.astype(jnp.bfloat16), Wp_ref[...],
                            preferred_element_type=jnp.float32)
                    + bp_ref[...])


def _run(kx, ky, scores, W0, W1, W2, b0r, b1r, b2r, WeAB, ber, Wp, bpr,
         block_n, interpret=False):
    n = kx.shape[0]
    grid = (n // block_n,)
    rep2 = lambda i: (0, 0)
    return pl.pallas_call(
        _body,
        grid=grid,
        in_specs=[
            pl.BlockSpec((block_n, 17), lambda i: (i, 0)),
            pl.BlockSpec((block_n, 17), lambda i: (i, 0)),
            pl.BlockSpec((block_n, 17), lambda i: (i, 0)),
            pl.BlockSpec((3, 64), rep2),
            pl.BlockSpec((3, 64), rep2),
            pl.BlockSpec((3, 64), rep2),
            pl.BlockSpec((1, 64), rep2),
            pl.BlockSpec((1, 64), rep2),
            pl.BlockSpec((1, 64), rep2),
            pl.BlockSpec((64, 128), rep2),
            pl.BlockSpec((1, 64), rep2),
            pl.BlockSpec((384, 128), rep2),
            pl.BlockSpec((1, 128), rep2),
        ],
        out_specs=pl.BlockSpec((block_n, 128), lambda i: (i, 0)),
        out_shape=jax.ShapeDtypeStruct((n, 128), jnp.float32),
        compiler_params=pltpu.CompilerParams(
            dimension_semantics=("parallel",)),
        interpret=interpret,
    )(kx, ky, scores, W0, W1, W2, b0r, b1r, b2r, WeAB, ber, Wp, bpr)


def kernel(keypoints, scores, W0, b0, W1, b1, W2, b2, We, be, Wp, bp,
           interpret=False):
    kx = keypoints[:, :, 0]
    ky = keypoints[:, :, 1]
    WeAB = jnp.concatenate([We[:64] - We[64:], We[64:]],
                           axis=1).astype(jnp.bfloat16)
    return _run(kx, ky, scores, W0, W1, W2,
                b0.reshape(1, 64), b1.reshape(1, 64), b2.reshape(1, 64),
                WeAB, be.reshape(1, 64), Wp.astype(jnp.bfloat16),
                bp.reshape(1, 128), block_n=512, interpret=interpret)


# speedup vs baseline: 34.5112x; 1.0682x over previous
import numpy as np
import jax
import jax.numpy as jnp
from jax import lax
from jax.experimental import pallas as pl
from jax.experimental.pallas import tpu as pltpu

_S0 = (0, 5, 6, 11, 12)
_S1 = (7, 8, 13, 14)
_S2 = (9, 10, 15, 16)
_SUBSETS = (_S0, _S1, _S2)
_CROSS_COUNT = (float(len(_S1)), float(len(_S0) + len(_S2)), float(len(_S1)))
_JLVL = ([(u, 0) for u in _S0] + [(u, 1) for u in _S1] + [(u, 2) for u in _S2])
_NJ13 = len(_JLVL)

_PX = np.zeros((_NJ13, 72), np.float32)
_PY = np.zeros((_NJ13, 72), np.float32)
_PS = np.zeros((_NJ13, 72), np.float32)
_PB = np.zeros((_NJ13, 72), np.float32)
_LSEL = np.zeros((_NJ13, 3), np.float32)
for _jj, (_u, _l) in enumerate(_JLVL):
    _PX[_jj, _u] = 1.0
    _PY[_jj, 24 + _u] = 1.0
    _PS[_jj, 48 + _u] = 1.0
    _PB[_jj, 65] = 1.0
    _LSEL[_jj, _l] = 1.0
_SPART = np.repeat(_PS, 64, axis=0)


def _body(kt_ref, wbig_ref, weabt_ref, bet_ref, wp_ref, bp_ref, out_ref):
    kt = kt_ref[...]
    nb = kt.shape[1]

    kx = kt[0:17]
    ky = kt[24:41]
    xmin = jnp.min(kx, axis=0, keepdims=True)
    xmax = jnp.max(kx, axis=0, keepdims=True)
    ymin = jnp.min(ky, axis=0, keepdims=True)
    ymax = jnp.max(ky, axis=0, keepdims=True)
    xnb = (kt[0:24] - xmin) / (xmax - xmin + 1e-6)
    ynb = (kt[24:48] - ymin) / (ymax - ymin + 1e-6)
    gn = jnp.concatenate([xnb, ynb, kt[48:72]],
                         axis=0).astype(jnp.bfloat16)

    hs = jnp.dot(wbig_ref[...], gn,
                 preferred_element_type=jnp.float32)
    h = hs[0:832]
    sb = hs[832:1664]
    m = jnp.maximum(h, 0.0) * sb

    totals = []
    feats = []
    off = 0
    for li, S in enumerate(_SUBSETS):
        n = len(S)
        ms = [m[(off + idx) * 64:(off + idx + 1) * 64] for idx in range(n)]
        sbs = [sb[(off + idx) * 64:(off + idx + 1) * 64] for idx in range(n)]
        off += n
        total = ms[0]
        for mm in ms[1:]:
            total = total + mm
        totals.append(total)
        for idx in range(n):
            feats.append(((total - ms[idx]) * sbs[idx]).astype(jnp.bfloat16))

    fcat = jnp.concatenate(feats, axis=1)
    fcat = jnp.concatenate(
        [fcat, jnp.ones((1, _NJ13 * nb), jnp.bfloat16)], axis=0)
    uv = jnp.dot(weabt_ref[...], fcat,
                 preferred_element_type=jnp.float32)

    bet = bet_ref[...]
    pooled_parts = []
    off = 0
    for li, S in enumerate(_SUBSETS):
        n = len(S)
        us = [uv[0:64, (off + idx) * nb:(off + idx + 1) * nb]
              for idx in range(n)]
        vs = [uv[64:128, (off + idx) * nb:(off + idx + 1) * nb]
              for idx in range(n)]
        off += n
        maxall = vs[0]
        for v in vs[1:]:
            maxall = jnp.maximum(maxall, v)
        zsum = None
        for idx in range(n):
            om = None
            for j2 in range(n):
                if j2 == idx:
                    continue
                om = vs[j2] if om is None else jnp.maximum(om, vs[j2])
            z = jnp.maximum(us[idx] + jnp.maximum(om, 0.0), 0.0)
            zsum = z if zsum is None else zsum + z
        zc = jnp.maximum(bet + maxall, 0.0)
        mean_z = (zsum + _CROSS_COUNT[li] * zc) * (1.0 / 17.0)
        mean_h = totals[li] * ((n - 1) / 17.0)
        pooled_parts.append(mean_h)
        pooled_parts.append(mean_z)

    poolt = jnp.concatenate(pooled_parts, axis=0)
    out = lax.dot_general(poolt.astype(jnp.bfloat16), wp_ref[...],
                          (((0,), (0,)), ((), ())),
                          preferred_element_type=jnp.float32)
    out_ref[...] = out + bp_ref[...]


def kernel(keypoints, scores, W0, b0, W1, b1, W2, b2, We, be, Wp, bp,
           interpret=False):
    n = keypoints.shape[0]
    f32 = jnp.float32

    kt = jnp.concatenate([
        keypoints[:, :, 0].T, jnp.zeros((7, n), f32),
        keypoints[:, :, 1].T, jnp.zeros((7, n), f32),
        scores.T, jnp.ones((1, n), f32), jnp.zeros((6, n), f32)], axis=0)

    wst = jnp.stack([W0, W1, W2])
    bst = jnp.stack([b0, b1, b2])
    lsel = jnp.asarray(_LSEL)
    wx = lsel @ wst[:, 0, :]
    wy = lsel @ wst[:, 1, :]
    ws = lsel @ wst[:, 2, :]
    bj = lsel @ bst
    hpart = (wx[:, :, None] * jnp.asarray(_PX)[:, None, :]
             + wy[:, :, None] * jnp.asarray(_PY)[:, None, :]
             + ws[:, :, None] * jnp.asarray(_PS)[:, None, :]
             + bj[:, :, None] * jnp.asarray(_PB)[:, None, :])
    wbig = jnp.concatenate([hpart.reshape(832, 72),
                            jnp.asarray(_SPART)], axis=0).astype(jnp.bfloat16)

    wea = We[:64]
    web = We[64:]
    weabt = jnp.concatenate([
        jnp.concatenate([(wea - web).T, be[:, None]], axis=1),
        jnp.concatenate([web.T, jnp.zeros((64, 1), f32)], axis=1)],
        axis=0).astype(jnp.bfloat16)

    block_n = 512
    grid = (n // block_n,)
    rep = lambda i: (0, 0)
    return pl.pallas_call(
        _body,
        grid=grid,
        in_specs=[
            pl.BlockSpec((72, block_n), lambda i: (0, i)),
            pl.BlockSpec((1664, 72), rep),
            pl.BlockSpec((128, 65), rep),
            pl.BlockSpec((64, 1), rep),
            pl.BlockSpec((384, 128), rep),
            pl.BlockSpec((1, 128), rep),
        ],
        out_specs=pl.BlockSpec((block_n, 128), lambda i: (i, 0)),
        out_shape=jax.ShapeDtypeStruct((n, 128), jnp.float32),
        compiler_params=pltpu.CompilerParams(
            dimension_semantics=("parallel",)),
        interpret=interpret,
    )(kt, wbig, weabt, be[:, None], Wp.astype(jnp.bfloat16),
      bp.reshape(1, 128))


# speedup vs baseline: 62.4173x; 1.8086x over previous
import jax
import jax.numpy as jnp
from jax import lax
from jax.experimental import pallas as pl
from jax.experimental.pallas import tpu as pltpu

def _deint_mat():
    rows = lax.broadcasted_iota(jnp.int32, (34, 34), 0)
    cols = lax.broadcasted_iota(jnp.int32, (34, 34), 1)
    ind_x = 1 - jnp.minimum(jnp.abs(cols - 2 * rows), 1)
    ind_y = 1 - jnp.minimum(jnp.abs(cols - 2 * rows + 33), 1)
    is_lt17 = jnp.minimum(jnp.maximum(17 - rows, 0), 1)
    return (ind_x * is_lt17 + ind_y * (1 - is_lt17)).astype(jnp.float32)

_S0 = (0, 5, 6, 11, 12)
_S1 = (7, 8, 13, 14)
_S2 = (9, 10, 15, 16)
_SUBSETS = (_S0, _S1, _S2)
_CROSS_COUNT = (float(len(_S1)), float(len(_S0) + len(_S2)), float(len(_S1)))
_NJ13 = sum(len(s) for s in _SUBSETS)


def _body(k34_ref, s_ref, W0_ref, W1_ref, W2_ref, b0_ref, b1_ref, b2_ref,
          We_ref, be_ref, Wp_ref, bp_ref, out_ref):
    bf16 = jnp.bfloat16
    f32 = jnp.float32
    nb = k34_ref.shape[0]

    t34 = jnp.transpose(k34_ref[...])
    st = jnp.transpose(s_ref[...])
    kxy = jnp.dot(_deint_mat(), t34,
                  preferred_element_type=f32)
    kxt = kxy[0:17]
    kyt = kxy[17:34]
    xmin = jnp.min(kxt, axis=0, keepdims=True)
    xmax = jnp.max(kxt, axis=0, keepdims=True)
    ymin = jnp.min(kyt, axis=0, keepdims=True)
    ymax = jnp.max(kyt, axis=0, keepdims=True)
    xn = ((kxt - xmin) / (xmax - xmin + 1e-6)).astype(bf16)
    yn = ((kyt - ymin) / (ymax - ymin + 1e-6)).astype(bf16)
    stb = st.astype(bf16)
    ones_row = jnp.ones((1, nb), bf16)

    w_refs = (W0_ref, W1_ref, W2_ref)
    b_refs = (b0_ref, b1_ref, b2_ref)
    srow_sel = jnp.concatenate(
        [jnp.zeros((64, 2), bf16), jnp.ones((64, 1), bf16),
         jnp.zeros((64, 1), bf16)], axis=1)
    totals = []
    feats = []
    for li, S in enumerate(_SUBSETS):
        p = jnp.concatenate(
            [jnp.concatenate([xn[u:u + 1], yn[u:u + 1], stb[u:u + 1],
                              ones_row], axis=0) for u in S],
            axis=1)
        wt = jnp.concatenate(
            [jnp.transpose(w_refs[li][...]),
             jnp.transpose(b_refs[li][...])], axis=1).astype(bf16)
        wcat = jnp.concatenate([wt, srow_sel], axis=0)
        hs = jnp.dot(wcat, p, preferred_element_type=f32)
        ms = []
        sbs = []
        for idx in range(len(S)):
            blk = hs[:, idx * nb:(idx + 1) * nb]
            sb = blk[64:128]
            ms.append(jnp.maximum(blk[0:64], 0.0) * sb)
            sbs.append(sb)
        total = ms[0]
        for mm in ms[1:]:
            total = total + mm
        totals.append(total)
        for idx in range(len(S)):
            feats.append(((total - ms[idx]) * sbs[idx]).astype(bf16))

    fcat = jnp.concatenate(feats, axis=1)
    fcat = jnp.concatenate(
        [fcat, jnp.ones((1, _NJ13 * nb), bf16)], axis=0)
    wea = We_ref[0:64, :]
    web = We_ref[64:128, :]
    bet = jnp.transpose(be_ref[...])
    weabt = jnp.concatenate([
        jnp.concatenate([jnp.transpose(wea - web), bet], axis=1),
        jnp.concatenate([jnp.transpose(web), jnp.zeros((64, 1), f32)],
                        axis=1)], axis=0).astype(bf16)
    uv = jnp.dot(weabt, fcat, preferred_element_type=f32)

    pooled_parts = []
    off = 0
    for li, S in enumerate(_SUBSETS):
        n = len(S)
        us = [uv[0:64, (off + idx) * nb:(off + idx + 1) * nb]
              for idx in range(n)]
        vs = [uv[64:128, (off + idx) * nb:(off + idx + 1) * nb]
              for idx in range(n)]
        off += n
        maxall = vs[0]
        for v in vs[1:]:
            maxall = jnp.maximum(maxall, v)
        zsum = None
        for idx in range(n):
            om = None
            for j2 in range(n):
                if j2 == idx:
                    continue
                om = vs[j2] if om is None else jnp.maximum(om, vs[j2])
            z = jnp.maximum(us[idx] + jnp.maximum(om, 0.0), 0.0)
            zsum = z if zsum is None else zsum + z
        zc = jnp.maximum(bet + maxall, 0.0)
        mean_z = (zsum + _CROSS_COUNT[li] * zc) * (1.0 / 17.0)
        mean_h = totals[li] * ((n - 1) / 17.0)
        pooled_parts.append(mean_h)
        pooled_parts.append(mean_z)

    poolt = jnp.concatenate(pooled_parts, axis=0)
    out = lax.dot_general(poolt.astype(bf16), Wp_ref[...].astype(bf16),
                          (((0,), (0,)), ((), ())),
                          preferred_element_type=f32)
    out_ref[...] = out + bp_ref[...]


def kernel(keypoints, scores, W0, b0, W1, b1, W2, b2, We, be, Wp, bp,
           interpret=False):
    n = keypoints.shape[0]
    k34 = keypoints.reshape(n, 34)
    block_n = 512
    grid = (n // block_n,)
    rep = lambda i: (0, 0)
    return pl.pallas_call(
        _body,
        grid=grid,
        in_specs=[
            pl.BlockSpec((block_n, 34), lambda i: (i, 0)),
            pl.BlockSpec((block_n, 17), lambda i: (i, 0)),
            pl.BlockSpec((3, 64), rep),
            pl.BlockSpec((3, 64), rep),
            pl.BlockSpec((3, 64), rep),
            pl.BlockSpec((1, 64), rep),
            pl.BlockSpec((1, 64), rep),
            pl.BlockSpec((1, 64), rep),
            pl.BlockSpec((128, 64), rep),
            pl.BlockSpec((1, 64), rep),
            pl.BlockSpec((384, 128), rep),
            pl.BlockSpec((1, 128), rep),
        ],
        out_specs=pl.BlockSpec((block_n, 128), lambda i: (i, 0)),
        out_shape=jax.ShapeDtypeStruct((n, 128), jnp.float32),
        compiler_params=pltpu.CompilerParams(
            dimension_semantics=("parallel",)),
        interpret=interpret,
    )(k34, scores, W0, W1, W2, b0.reshape(1, 64), b1.reshape(1, 64),
      b2.reshape(1, 64), We, be.reshape(1, 64), Wp, bp.reshape(1, 128))


# speedup vs baseline: 63.7065x; 1.0207x over previous
import jax
import jax.numpy as jnp
from jax import lax
from jax.experimental import pallas as pl
from jax.experimental.pallas import tpu as pltpu

def _deint_mat():
    rows = lax.broadcasted_iota(jnp.int32, (34, 34), 0)
    cols = lax.broadcasted_iota(jnp.int32, (34, 34), 1)
    ind_x = 1 - jnp.minimum(jnp.abs(cols - 2 * rows), 1)
    ind_y = 1 - jnp.minimum(jnp.abs(cols - 2 * rows + 33), 1)
    is_lt17 = jnp.minimum(jnp.maximum(17 - rows, 0), 1)
    return (ind_x * is_lt17 + ind_y * (1 - is_lt17)).astype(jnp.float32)

_S0 = (0, 5, 6, 11, 12)
_S1 = (7, 8, 13, 14)
_S2 = (9, 10, 15, 16)
_SUBSETS = (_S0, _S1, _S2)
_CROSS_COUNT = (float(len(_S1)), float(len(_S0) + len(_S2)), float(len(_S1)))
_NJ13 = sum(len(s) for s in _SUBSETS)


def _body(k34_ref, s_ref, W0_ref, W1_ref, W2_ref, b0_ref, b1_ref, b2_ref,
          We_ref, be_ref, Wp_ref, bp_ref, out_ref):
    bf16 = jnp.bfloat16
    f32 = jnp.float32
    nb = k34_ref.shape[0]

    t34 = jnp.transpose(k34_ref[...])
    st = jnp.transpose(s_ref[...])
    kxy = jnp.dot(_deint_mat(), t34,
                  preferred_element_type=f32)
    kxt = kxy[0:17]
    kyt = kxy[17:34]
    xmin = jnp.min(kxt, axis=0, keepdims=True)
    xmax = jnp.max(kxt, axis=0, keepdims=True)
    ymin = jnp.min(kyt, axis=0, keepdims=True)
    ymax = jnp.max(kyt, axis=0, keepdims=True)
    xn = ((kxt - xmin) / (xmax - xmin + 1e-6)).astype(bf16)
    yn = ((kyt - ymin) / (ymax - ymin + 1e-6)).astype(bf16)
    stb = st.astype(bf16)
    ones_row = jnp.ones((1, nb), bf16)

    w_refs = (W0_ref, W1_ref, W2_ref)
    b_refs = (b0_ref, b1_ref, b2_ref)
    srow_sel = jnp.concatenate(
        [jnp.zeros((64, 2), bf16), jnp.ones((64, 1), bf16),
         jnp.zeros((64, 1), bf16)], axis=1)
    totals = []
    feats = []
    for li, S in enumerate(_SUBSETS):
        p = jnp.concatenate(
            [jnp.concatenate([xn[u:u + 1], yn[u:u + 1], stb[u:u + 1],
                              ones_row], axis=0) for u in S],
            axis=1)
        wt = jnp.concatenate(
            [jnp.transpose(w_refs[li][...]),
             jnp.transpose(b_refs[li][...])], axis=1).astype(bf16)
        wcat = jnp.concatenate([wt, srow_sel], axis=0)
        hs = jnp.dot(wcat, p, preferred_element_type=f32)
        ms = []
        sbs = []
        for idx in range(len(S)):
            blk = hs[:, idx * nb:(idx + 1) * nb]
            sb = blk[64:128]
            ms.append(jnp.maximum(blk[0:64], 0.0) * sb)
            sbs.append(sb)
        total = ms[0]
        for mm in ms[1:]:
            total = total + mm
        totals.append(total)
        for idx in range(len(S)):
            feats.append(((total - ms[idx]) * sbs[idx]).astype(bf16))

    fcat = jnp.concatenate(feats, axis=1)
    fcat = jnp.concatenate(
        [fcat, jnp.ones((1, _NJ13 * nb), bf16)], axis=0)
    wea = We_ref[0:64, :]
    web = We_ref[64:128, :]
    bet = jnp.transpose(be_ref[...])
    weabt = jnp.concatenate([
        jnp.concatenate([jnp.transpose(wea - web), bet], axis=1),
        jnp.concatenate([jnp.transpose(web), jnp.zeros((64, 1), f32)],
                        axis=1)], axis=0).astype(bf16)
    uv = jnp.dot(weabt, fcat, preferred_element_type=f32)

    pooled_parts = []
    off = 0
    for li, S in enumerate(_SUBSETS):
        n = len(S)
        us = [uv[0:64, (off + idx) * nb:(off + idx + 1) * nb]
              for idx in range(n)]
        vs = [uv[64:128, (off + idx) * nb:(off + idx + 1) * nb]
              for idx in range(n)]
        off += n
        maxall = vs[0]
        for v in vs[1:]:
            maxall = jnp.maximum(maxall, v)
        zsum = None
        for idx in range(n):
            om = None
            for j2 in range(n):
                if j2 == idx:
                    continue
                om = vs[j2] if om is None else jnp.maximum(om, vs[j2])
            z = jnp.maximum(us[idx] + jnp.maximum(om, 0.0), 0.0)
            zsum = z if zsum is None else zsum + z
        zc = jnp.maximum(bet + maxall, 0.0)
        mean_z = (zsum + _CROSS_COUNT[li] * zc) * (1.0 / 17.0)
        mean_h = totals[li] * ((n - 1) / 17.0)
        pooled_parts.append(mean_h)
        pooled_parts.append(mean_z)

    poolt = jnp.concatenate(pooled_parts, axis=0)
    out = lax.dot_general(poolt.astype(bf16), Wp_ref[...].astype(bf16),
                          (((0,), (0,)), ((), ())),
                          preferred_element_type=f32)
    out_ref[...] = out + bp_ref[...]


def kernel(keypoints, scores, W0, b0, W1, b1, W2, b2, We, be, Wp, bp,
           interpret=False):
    n = keypoints.shape[0]
    k34 = keypoints.reshape(n, 34)
    block_n = 1024
    grid = (n // block_n,)
    rep = lambda i: (0, 0)
    return pl.pallas_call(
        _body,
        grid=grid,
        in_specs=[
            pl.BlockSpec((block_n, 34), lambda i: (i, 0)),
            pl.BlockSpec((block_n, 17), lambda i: (i, 0)),
            pl.BlockSpec((3, 64), rep),
            pl.BlockSpec((3, 64), rep),
            pl.BlockSpec((3, 64), rep),
            pl.BlockSpec((1, 64), rep),
            pl.BlockSpec((1, 64), rep),
            pl.BlockSpec((1, 64), rep),
            pl.BlockSpec((128, 64), rep),
            pl.BlockSpec((1, 64), rep),
            pl.BlockSpec((384, 128), rep),
            pl.BlockSpec((1, 128), rep),
        ],
        out_specs=pl.BlockSpec((block_n, 128), lambda i: (i, 0)),
        out_shape=jax.ShapeDtypeStruct((n, 128), jnp.float32),
        compiler_params=pltpu.CompilerParams(
            dimension_semantics=("parallel",)),
        interpret=interpret,
    )(k34, scores, W0, W1, W2, b0.reshape(1, 64), b1.reshape(1, 64),
      b2.reshape(1, 64), We, be.reshape(1, 64), Wp, bp.reshape(1, 128))
